# Initial kernel scaffold; baseline (speedup 1.0000x reference)
import jax
import jax.numpy as jnp
from jax.experimental import pallas as pl


def kernel(new_edges, recv_idx):
    raise NotImplementedError("write your pallas kernel here")


# speedup vs baseline: 4.4633x; 4.4633x over previous
import functools

import jax
import jax.numpy as jnp
from jax import lax
from jax.experimental import pallas as pl
from jax.experimental.pallas import tpu as pltpu
from jax.experimental.pallas import tpu_sc as plsc

N_NODES = 10000
N_EDGES = 320000
D = 128

NC = 2
NS = 16
NW = NC * NS

EPT = N_EDGES // NW
CHUNK = 80
NCHUNK = EPT // CHUNK
RPT = 640
RPT_LAST = N_NODES - 15 * RPT


def _sc_partial_sums(edges, idx3):
    mesh = plsc.VectorSubcoreMesh(core_axis_name="c", subcore_axis_name="s")

    @functools.partial(
        pl.kernel,
        out_type=jax.ShapeDtypeStruct((2 * N_NODES, D), jnp.float32),
        mesh=mesh,
        scratch_types=dict(
            acc=pltpu.VMEM_SHARED((N_NODES, D), jnp.float32),
            idx_v=pltpu.VMEM((NCHUNK, CHUNK), jnp.int32),
            ebuf=pltpu.VMEM((CHUNK, D), jnp.float32),
        ),
    )
    def k(edges_hbm, idx_hbm, out_hbm, acc, idx_v, ebuf):
        core = lax.axis_index("c")
        sub = lax.axis_index("s")
        wid = core * NS + sub

        def zero_row(r, carry):
            for j in range(D // 16):
                ebuf[r, pl.ds(j * 16, 16)] = jnp.zeros((16,), jnp.float32)
            return carry

        lax.fori_loop(0, CHUNK, zero_row, 0)

        @pl.when(sub < NS - 1)
        def _():
            for kk in range(RPT // CHUNK):
                pltpu.sync_copy(ebuf, acc.at[pl.ds(sub * RPT + kk * CHUNK, CHUNK)])

        @pl.when(sub == NS - 1)
        def _():
            for kk in range(RPT_LAST // CHUNK):
                pltpu.sync_copy(ebuf, acc.at[pl.ds(15 * RPT + kk * CHUNK, CHUNK)])

        plsc.subcore_barrier()

        pltpu.sync_copy(idx_hbm.at[wid], idx_v)

        def body(c, carry):
            row0 = wid * EPT + c * CHUNK
            pltpu.sync_copy(edges_hbm.at[pl.ds(row0, CHUNK)], ebuf)
            pltpu.sync_copy(ebuf, acc.at[idx_v.at[c]], add=True)
            return carry

        lax.fori_loop(0, NCHUNK, body, 0)
        plsc.subcore_barrier()

        @pl.when(sub < NS - 1)
        def _():
            pltpu.sync_copy(
                acc.at[pl.ds(sub * RPT, RPT)],
                out_hbm.at[pl.ds(core * N_NODES + sub * RPT, RPT)],
            )

        @pl.when(sub == NS - 1)
        def _():
            pltpu.sync_copy(
                acc.at[pl.ds(15 * RPT, RPT_LAST)],
                out_hbm.at[pl.ds(core * N_NODES + 15 * RPT, RPT_LAST)],
            )

    return k(edges, idx3)


def _merge_kernel(a_ref, b_ref, o_ref):
    o_ref[...] = a_ref[...] + b_ref[...]


def _tc_merge(partials):
    blk = 1000
    return pl.pallas_call(
        _merge_kernel,
        out_shape=jax.ShapeDtypeStruct((N_NODES, D), jnp.float32),
        grid=(N_NODES // blk,),
        in_specs=[
            pl.BlockSpec((blk, D), lambda i: (i, 0)),
            pl.BlockSpec((blk, D), lambda i: (i, 0)),
        ],
        out_specs=pl.BlockSpec((blk, D), lambda i: (i, 0)),
    )(partials[:N_NODES], partials[N_NODES:])


@jax.jit
def kernel(new_edges, recv_idx):
    idx3 = recv_idx.astype(jnp.int32).reshape(NW, NCHUNK, CHUNK)
    partials = _sc_partial_sums(new_edges, idx3)
    return _tc_merge(partials)


# speedup vs baseline: 7.4917x; 1.6785x over previous
import functools

import jax
import jax.numpy as jnp
from jax import lax
from jax.experimental import pallas as pl
from jax.experimental.pallas import tpu as pltpu
from jax.experimental.pallas import tpu_sc as plsc

N_NODES = 10000
N_EDGES = 320000
D = 128

NC = 2
NS = 16
NW = NC * NS

EPT = N_EDGES // NW
CHUNK = 80
NCHUNK = EPT // CHUNK
NBUF = 4
RPT = 640
RPT_LAST = N_NODES - 15 * RPT


def _sc_partial_sums(edges, idx3):
    mesh = plsc.VectorSubcoreMesh(core_axis_name="c", subcore_axis_name="s")

    @functools.partial(
        pl.kernel,
        out_type=jax.ShapeDtypeStruct((2 * N_NODES, D), jnp.float32),
        mesh=mesh,
        scratch_types=dict(
            acc=pltpu.VMEM_SHARED((N_NODES, D), jnp.float32),
            idx_g=pltpu.VMEM((NBUF, 1, CHUNK), jnp.int32),
            ebuf=pltpu.VMEM((NBUF, CHUNK, D), jnp.float32),
            sems=pltpu.SemaphoreType.DMA((NBUF,)),
        ),
    )
    def k(edges_hbm, idx_hbm, out_hbm, acc, idx_g, ebuf, sems):
        core = lax.axis_index("c")
        sub = lax.axis_index("s")
        wid = core * NS + sub

        def zero_row(r, carry):
            for j in range(D // 16):
                ebuf[0, r, pl.ds(j * 16, 16)] = jnp.zeros((16,), jnp.float32)
            return carry

        lax.fori_loop(0, CHUNK, zero_row, 0)

        @pl.when(sub < NS - 1)
        def _():
            for kk in range(RPT // CHUNK):
                pltpu.sync_copy(ebuf.at[0], acc.at[pl.ds(sub * RPT + kk * CHUNK, CHUNK)])

        @pl.when(sub == NS - 1)
        def _():
            for kk in range(RPT_LAST // CHUNK):
                pltpu.sync_copy(ebuf.at[0], acc.at[pl.ds(15 * RPT + kk * CHUNK, CHUNK)])

        plsc.subcore_barrier()

        def start_gather(c, b):
            row0 = pl.multiple_of(wid * EPT + c * CHUNK, 8)
            pltpu.async_copy(edges_hbm.at[pl.ds(row0, CHUNK)], ebuf.at[b], sems.at[b])
            pltpu.async_copy(idx_hbm.at[wid, c], idx_g.at[b], sems.at[b])

        def wait_gather(b):
            pltpu.make_async_copy(
                edges_hbm.at[pl.ds(0, CHUNK)], ebuf.at[b], sems.at[b]
            ).wait()
            pltpu.make_async_copy(
                idx_hbm.at[0, 0], idx_g.at[b], sems.at[b]
            ).wait()

        for b in range(NBUF):
            start_gather(b, b)

        n_main = (NCHUNK - NBUF - 1) // NBUF

        def body(i, carry):
            for b in range(NBUF):
                c = i * NBUF + b
                wait_gather(b)
                pltpu.sync_copy(ebuf.at[b], acc.at[idx_g.at[b, 0]], add=True)
                start_gather(c + NBUF, b)
            return carry

        lax.fori_loop(0, n_main, body, 0)

        done = n_main * NBUF
        for t in range(NCHUNK - done):
            b = t % NBUF
            c = done + t
            wait_gather(b)
            pltpu.sync_copy(ebuf.at[b], acc.at[idx_g.at[b, 0]], add=True)
            nxt = done + t + NBUF
            if nxt < NCHUNK:
                start_gather(nxt, b)

        plsc.subcore_barrier()

        @pl.when(sub < NS - 1)
        def _():
            pltpu.sync_copy(
                acc.at[pl.ds(sub * RPT, RPT)],
                out_hbm.at[pl.ds(core * N_NODES + sub * RPT, RPT)],
            )

        @pl.when(sub == NS - 1)
        def _():
            pltpu.sync_copy(
                acc.at[pl.ds(15 * RPT, RPT_LAST)],
                out_hbm.at[pl.ds(core * N_NODES + 15 * RPT, RPT_LAST)],
            )

    return k(edges, idx3)


def _merge_kernel(a_ref, b_ref, o_ref):
    o_ref[...] = a_ref[...] + b_ref[...]


def _tc_merge(partials):
    blk = 1000
    return pl.pallas_call(
        _merge_kernel,
        out_shape=jax.ShapeDtypeStruct((N_NODES, D), jnp.float32),
        grid=(N_NODES // blk,),
        in_specs=[
            pl.BlockSpec((blk, D), lambda i: (i, 0)),
            pl.BlockSpec((blk, D), lambda i: (i, 0)),
        ],
        out_specs=pl.BlockSpec((blk, D), lambda i: (i, 0)),
    )(partials[:N_NODES], partials[N_NODES:])


@jax.jit
def kernel(new_edges, recv_idx):
    idx3 = recv_idx.astype(jnp.int32).reshape(NW, NCHUNK, 1, CHUNK)
    partials = _sc_partial_sums(new_edges, idx3)
    return _tc_merge(partials)
